# baseline (speedup 1.0000x reference)
import functools

import jax
import jax.numpy as jnp
from jax import lax
from jax.experimental import pallas as pl
from jax.experimental.pallas import tpu as pltpu
from jax.experimental.pallas import tpu_sc as plsc

NC = 2
NS = 16
NW = NC * NS
K = 128
N_PAD_ALIGN = NS * 8


def _pad_nodes(n):
    per = -(-n // NS)
    per = -(-per // 8) * 8
    return per * NS, per


def _sc_degree(row3, w3, n_nodes, n_chunks):
    n_pad, per_sub = _pad_nodes(n_nodes)
    mesh = plsc.VectorSubcoreMesh(core_axis_name="c", subcore_axis_name="s")

    @functools.partial(
        pl.kernel,
        out_type=jax.ShapeDtypeStruct((NC * n_pad,), jnp.float32),
        mesh=mesh,
        scratch_types=[
            pltpu.VMEM((n_chunks, K), jnp.int32),
            pltpu.VMEM((n_chunks, K), jnp.float32),
            pltpu.VMEM((per_sub,), jnp.float32),
            pltpu.VMEM_SHARED((n_pad,), jnp.float32),
        ],
    )
    def deg_kernel(row_hbm, w_hbm, out_hbm, idx_v, w_v, z_v, acc_sh):
        c = lax.axis_index("c")
        s = lax.axis_index("s")
        wid = s * NC + c
        pltpu.sync_copy(row_hbm.at[wid], idx_v)
        pltpu.sync_copy(w_hbm.at[wid], w_v)

        def zero_body(i, carry):
            z_v[pl.ds(i * 16, 16)] = jnp.zeros((16,), jnp.float32)
            return carry

        lax.fori_loop(0, per_sub // 16, zero_body, 0)
        pltpu.sync_copy(z_v, acc_sh.at[pl.ds(s * per_sub, per_sub)])
        plsc.subcore_barrier()

        def chunk_body(j, carry):
            pltpu.sync_copy(w_v.at[j], acc_sh.at[idx_v.at[j]], add=True)
            return carry

        lax.fori_loop(0, n_chunks, chunk_body, 0)
        plsc.subcore_barrier()
        pltpu.sync_copy(acc_sh.at[pl.ds(s * per_sub, per_sub)], z_v)
        pltpu.sync_copy(z_v, out_hbm.at[pl.ds(c * n_pad + s * per_sub, per_sub)])

    return deg_kernel(row3, w3)


def _sc_aggregate(hp, row3, col3, w3, n_nodes, n_chunks, h_dim):
    n_pad, per_sub = _pad_nodes(n_nodes)
    mesh = plsc.VectorSubcoreMesh(core_axis_name="c", subcore_axis_name="s")
    nf = h_dim // 16
    nbuf = 4
    assert n_chunks % nbuf == 0 and n_chunks >= 2 * nbuf
    zrows = per_sub // 8

    @functools.partial(
        pl.kernel,
        out_type=jax.ShapeDtypeStruct((NC, n_pad, h_dim), jnp.float32),
        mesh=mesh,
        scratch_types=(
            [pltpu.VMEM((n_chunks, K), jnp.int32),
             pltpu.VMEM((n_chunks, K), jnp.int32),
             pltpu.VMEM((n_chunks, K), jnp.float32)]
            + [pltpu.VMEM((K, h_dim), jnp.float32) for _ in range(nbuf)]
            + [pltpu.VMEM((zrows, h_dim), jnp.float32)]
            + [pltpu.VMEM_SHARED((n_pad, h_dim), jnp.float32)]
            + [pltpu.SemaphoreType.DMA for _ in range(2 * nbuf)]
        ),
        compiler_params=pltpu.CompilerParams(use_tc_tiling_on_sc=False),
    )
    def agg_kernel(hp_hbm, row_hbm, col_hbm, w_hbm, out_hbm, *refs):
        ridx_v, cidx_v, w_v = refs[0], refs[1], refs[2]
        bufs = refs[3:3 + nbuf]
        z_v = refs[3 + nbuf]
        acc_sh = refs[4 + nbuf]
        gsem = refs[5 + nbuf:5 + 2 * nbuf]
        ssem = refs[5 + 2 * nbuf:5 + 3 * nbuf]
        c = lax.axis_index("c")
        s = lax.axis_index("s")
        wid = s * NC + c
        pltpu.sync_copy(row_hbm.at[wid], ridx_v)
        pltpu.sync_copy(col_hbm.at[wid], cidx_v)
        pltpu.sync_copy(w_hbm.at[wid], w_v)

        def zero_body(i, carry):
            for f in range(nf):
                z_v[i, pl.ds(f * 16, 16)] = jnp.zeros((16,), jnp.float32)
            return carry

        lax.fori_loop(0, zrows, zero_body, 0)
        for part in range(8):
            pltpu.sync_copy(
                z_v, acc_sh.at[pl.ds(s * per_sub + part * zrows, zrows)])
        plsc.subcore_barrier()

        def scale_buf(j, buf):
            def group_body(g, carry2):
                wv = w_v[j, pl.ds(g * 16, 16)]
                for e in range(16):
                    we = wv[e]
                    for f in range(nf):
                        buf[g * 16 + e, pl.ds(f * 16, 16)] = (
                            buf[g * 16 + e, pl.ds(f * 16, 16)] * we)
                return carry2

            lax.fori_loop(0, K // 16, group_body, 0)

        for b in range(nbuf):
            pltpu.async_copy(hp_hbm.at[cidx_v.at[b]], bufs[b], gsem[b])

        def group_loop(g, carry):
            for b in range(nbuf):
                j = g * nbuf + b
                bp = (b - 1) % nbuf
                pltpu.make_async_copy(hp_hbm.at[cidx_v.at[j]], bufs[b],
                                      gsem[b]).wait()
                scale_buf(j, bufs[b])

                @pl.when(j >= 1)
                def _drain_prev():
                    pltpu.make_async_copy(
                        bufs[bp], acc_sh.at[ridx_v.at[j - 1]],
                        ssem[bp]).wait()

                    @pl.when(j + 3 < n_chunks)
                    def _refill():
                        pltpu.async_copy(hp_hbm.at[cidx_v.at[j + 3]],
                                         bufs[bp], gsem[bp])

                pltpu.async_copy(bufs[b], acc_sh.at[ridx_v.at[j]],
                                 ssem[b], add=True)

            return carry

        lax.fori_loop(0, n_chunks // nbuf, group_loop, 0)
        pltpu.make_async_copy(bufs[(n_chunks - 1) % nbuf],
                              acc_sh.at[ridx_v.at[n_chunks - 1]],
                              ssem[(n_chunks - 1) % nbuf]).wait()
        plsc.subcore_barrier()
        for part in range(8):
            pltpu.sync_copy(
                acc_sh.at[pl.ds(s * per_sub + part * zrows, zrows)], z_v)
            pltpu.sync_copy(
                z_v, out_hbm.at[c, pl.ds(s * per_sub + part * zrows, zrows)])

    return agg_kernel(hp, row3, col3, w3)


def _tc_matmul(x, w):
    def body(x_ref, w_ref, o_ref):
        o_ref[...] = jnp.dot(x_ref[...], w_ref[...],
                             preferred_element_type=jnp.float32)

    return pl.pallas_call(
        body,
        out_shape=jax.ShapeDtypeStruct((x.shape[0], w.shape[1]), jnp.float32),
    )(x, w)


def _tc_scale(degp, h, n_nodes, n_pad):
    def body(degp_ref, h_ref, hp_ref, dis_ref):
        deg = (degp_ref[pl.ds(0, n_nodes)]
               + degp_ref[pl.ds(n_pad, n_nodes)] + 1.0)
        dis = lax.rsqrt(deg)
        dis_ref[...] = dis[:, None]
        hp_ref[...] = h_ref[...] * dis[:, None]

    return pl.pallas_call(
        body,
        out_shape=(
            jax.ShapeDtypeStruct(h.shape, jnp.float32),
            jax.ShapeDtypeStruct((n_nodes, 1), jnp.float32),
        ),
    )(degp, h)


def _tc_layer_out(aggp, hp, dis, b, n_nodes, w_next=None):
    h_dim = hp.shape[1]

    if w_next is None:
        def body(aggp_ref, hp_ref, dis_ref, b_ref, o_ref):
            agg = aggp_ref[0, :n_nodes] + aggp_ref[1, :n_nodes]
            o_ref[...] = jax.nn.relu(
                dis_ref[...] * (agg + hp_ref[...]) + b_ref[...])

        return pl.pallas_call(
            body,
            out_shape=jax.ShapeDtypeStruct((n_nodes, h_dim), jnp.float32),
        )(aggp, hp, dis, b[None, :])

    def body2(aggp_ref, hp_ref, dis_ref, b_ref, w_ref, o_ref):
        agg = aggp_ref[0, :n_nodes] + aggp_ref[1, :n_nodes]
        out = jax.nn.relu(dis_ref[...] * (agg + hp_ref[...]) + b_ref[...])
        o_ref[...] = jnp.dot(out, w_ref[...],
                             preferred_element_type=jnp.float32) * dis_ref[...]

    return pl.pallas_call(
        body2,
        out_shape=jax.ShapeDtypeStruct((n_nodes, w_next.shape[1]), jnp.float32),
    )(aggp, hp, dis, b[None, :], w_next)


def kernel(x, edge_index, edge_weight, W0, b0, W1, b1):
    n_nodes = x.shape[0]
    n_edges = edge_weight.shape[0]
    n_chunks = -(-n_edges // (NW * K))
    n_chunks = -(-n_chunks // 4) * 4
    e_pad = NW * K * n_chunks

    row = edge_index[0].astype(jnp.int32)
    col = edge_index[1].astype(jnp.int32)
    w = edge_weight.astype(jnp.float32)
    pad = e_pad - n_edges
    if pad:
        row = jnp.concatenate([row, jnp.zeros((pad,), jnp.int32)])
        col = jnp.concatenate([col, jnp.zeros((pad,), jnp.int32)])
        w = jnp.concatenate([w, jnp.zeros((pad,), jnp.float32)])
    row3 = row.reshape(NW, n_chunks, K)
    col3 = col.reshape(NW, n_chunks, K)
    w3 = w.reshape(NW, n_chunks, K)

    degp = _sc_degree(row3, w3, n_nodes, n_chunks)
    h0 = _tc_matmul(x, W0)

    n_pad, _ = _pad_nodes(n_nodes)
    h0p, dis = _tc_scale(degp, h0, n_nodes, n_pad)
    agg0 = _sc_aggregate(h0p, row3, col3, w3, n_nodes, n_chunks, W0.shape[1])
    h1p = _tc_layer_out(agg0, h0p, dis, b0, n_nodes, w_next=W1)
    agg1 = _sc_aggregate(h1p, row3, col3, w3, n_nodes, n_chunks, W1.shape[1])
    return _tc_layer_out(agg1, h1p, dis, b1, n_nodes)


# speedup vs baseline: 1.4023x; 1.4023x over previous
import functools

import jax
import jax.numpy as jnp
from jax import lax
from jax.experimental import pallas as pl
from jax.experimental.pallas import tpu as pltpu
from jax.experimental.pallas import tpu_sc as plsc

NC = 2
NS = 16
NW = NC * NS
K = 128
N_PAD_ALIGN = NS * 8


def _pad_nodes(n):
    per = -(-n // NS)
    per = -(-per // 8) * 8
    return per * NS, per


def _sc_degree(row3, w3, n_nodes, n_chunks):
    n_pad, per_sub = _pad_nodes(n_nodes)
    mesh = plsc.VectorSubcoreMesh(core_axis_name="c", subcore_axis_name="s")

    @functools.partial(
        pl.kernel,
        out_type=jax.ShapeDtypeStruct((NC * n_pad,), jnp.float32),
        mesh=mesh,
        scratch_types=[
            pltpu.VMEM((n_chunks, K), jnp.int32),
            pltpu.VMEM((n_chunks, K), jnp.float32),
            pltpu.VMEM((per_sub,), jnp.float32),
            pltpu.VMEM_SHARED((n_pad,), jnp.float32),
        ],
    )
    def deg_kernel(row_hbm, w_hbm, out_hbm, idx_v, w_v, z_v, acc_sh):
        c = lax.axis_index("c")
        s = lax.axis_index("s")
        wid = s * NC + c
        pltpu.sync_copy(row_hbm.at[wid], idx_v)
        pltpu.sync_copy(w_hbm.at[wid], w_v)

        def zero_body(i, carry):
            z_v[pl.ds(i * 16, 16)] = jnp.zeros((16,), jnp.float32)
            return carry

        lax.fori_loop(0, per_sub // 16, zero_body, 0)
        pltpu.sync_copy(z_v, acc_sh.at[pl.ds(s * per_sub, per_sub)])
        plsc.subcore_barrier()

        def chunk_body(j, carry):
            pltpu.sync_copy(w_v.at[j], acc_sh.at[idx_v.at[j]], add=True)
            return carry

        lax.fori_loop(0, n_chunks, chunk_body, 0)
        plsc.subcore_barrier()
        pltpu.sync_copy(acc_sh.at[pl.ds(s * per_sub, per_sub)], z_v)
        pltpu.sync_copy(z_v, out_hbm.at[pl.ds(c * n_pad + s * per_sub, per_sub)])

    return deg_kernel(row3, w3)


def _sc_aggregate(hpa, hpb, row3, col3, w3, n_nodes, n_chunks, h_dim):
    n_pad, per_sub = _pad_nodes(n_nodes)
    mesh = plsc.VectorSubcoreMesh(core_axis_name="c", subcore_axis_name="s")
    h2 = h_dim // 2
    nf = h2 // 16
    nbuf = 4
    assert n_chunks % nbuf == 0 and n_chunks >= 2 * nbuf
    zrows = per_sub // 8

    @functools.partial(
        pl.kernel,
        out_type=jax.ShapeDtypeStruct((NC, 2, n_pad, h2), jnp.float32),
        mesh=mesh,
        scratch_types=(
            [pltpu.VMEM((n_chunks, K), jnp.int32),
             pltpu.VMEM((n_chunks, K), jnp.int32),
             pltpu.VMEM((n_chunks, K), jnp.float32)]
            + [pltpu.VMEM((K, h2), jnp.float32) for _ in range(2 * nbuf)]
            + [pltpu.VMEM((zrows, h2), jnp.float32)]
            + [pltpu.VMEM_SHARED((n_pad, h2), jnp.float32) for _ in range(2)]
            + [pltpu.SemaphoreType.DMA for _ in range(4 * nbuf)]
        ),
        compiler_params=pltpu.CompilerParams(use_tc_tiling_on_sc=False),
    )
    def agg_kernel(hpa_hbm, hpb_hbm, row_hbm, col_hbm, w_hbm, out_hbm, *refs):
        ridx_v, cidx_v, w_v = refs[0], refs[1], refs[2]
        bufsa = refs[3:3 + nbuf]
        bufsb = refs[3 + nbuf:3 + 2 * nbuf]
        z_v = refs[3 + 2 * nbuf]
        acca = refs[4 + 2 * nbuf]
        accb = refs[5 + 2 * nbuf]
        sems = refs[6 + 2 * nbuf:]
        gsema = sems[0:nbuf]
        gsemb = sems[nbuf:2 * nbuf]
        ssema = sems[2 * nbuf:3 * nbuf]
        ssemb = sems[3 * nbuf:4 * nbuf]
        c = lax.axis_index("c")
        s = lax.axis_index("s")
        wid = s * NC + c
        pltpu.sync_copy(row_hbm.at[wid], ridx_v)
        pltpu.sync_copy(col_hbm.at[wid], cidx_v)
        pltpu.sync_copy(w_hbm.at[wid], w_v)

        def zero_body(i, carry):
            for f in range(nf):
                z_v[i, pl.ds(f * 16, 16)] = jnp.zeros((16,), jnp.float32)
            return carry

        lax.fori_loop(0, zrows, zero_body, 0)
        for part in range(8):
            pltpu.sync_copy(
                z_v, acca.at[pl.ds(s * per_sub + part * zrows, zrows)])
            pltpu.sync_copy(
                z_v, accb.at[pl.ds(s * per_sub + part * zrows, zrows)])
        plsc.subcore_barrier()

        def scale_buf(j, buf):
            def group_body(g, carry2):
                wv = w_v[j, pl.ds(g * 16, 16)]
                for e in range(16):
                    we = wv[e]
                    for f in range(nf):
                        buf[g * 16 + e, pl.ds(f * 16, 16)] = (
                            buf[g * 16 + e, pl.ds(f * 16, 16)] * we)
                return carry2

            lax.fori_loop(0, K // 16, group_body, 0)

        for b in range(nbuf):
            pltpu.async_copy(hpa_hbm.at[cidx_v.at[b]], bufsa[b], gsema[b])
            pltpu.async_copy(hpb_hbm.at[cidx_v.at[b]], bufsb[b], gsemb[b])

        def group_loop(g, carry):
            for b in range(nbuf):
                j = g * nbuf + b
                bp = (b - 1) % nbuf
                pltpu.make_async_copy(hpa_hbm.at[cidx_v.at[j]], bufsa[b],
                                      gsema[b]).wait()
                scale_buf(j, bufsa[b])
                pltpu.make_async_copy(hpb_hbm.at[cidx_v.at[j]], bufsb[b],
                                      gsemb[b]).wait()
                scale_buf(j, bufsb[b])

                @pl.when(j >= 1)
                def _drain_prev():
                    pltpu.make_async_copy(
                        bufsa[bp], acca.at[ridx_v.at[j - 1]],
                        ssema[bp]).wait()
                    pltpu.make_async_copy(
                        bufsb[bp], accb.at[ridx_v.at[j - 1]],
                        ssemb[bp]).wait()

                    @pl.when(j + 3 < n_chunks)
                    def _refill():
                        pltpu.async_copy(hpa_hbm.at[cidx_v.at[j + 3]],
                                         bufsa[bp], gsema[bp])
                        pltpu.async_copy(hpb_hbm.at[cidx_v.at[j + 3]],
                                         bufsb[bp], gsemb[bp])

                pltpu.async_copy(bufsa[b], acca.at[ridx_v.at[j]],
                                 ssema[b], add=True)
                pltpu.async_copy(bufsb[b], accb.at[ridx_v.at[j]],
                                 ssemb[b], add=True)

            return carry

        lax.fori_loop(0, n_chunks // nbuf, group_loop, 0)
        bl = (n_chunks - 1) % nbuf
        pltpu.make_async_copy(bufsa[bl], acca.at[ridx_v.at[n_chunks - 1]],
                              ssema[bl]).wait()
        pltpu.make_async_copy(bufsb[bl], accb.at[ridx_v.at[n_chunks - 1]],
                              ssemb[bl]).wait()
        plsc.subcore_barrier()
        for part in range(8):
            pltpu.sync_copy(
                acca.at[pl.ds(s * per_sub + part * zrows, zrows)], z_v)
            pltpu.sync_copy(
                z_v, out_hbm.at[c, 0, pl.ds(s * per_sub + part * zrows, zrows)])
            pltpu.sync_copy(
                accb.at[pl.ds(s * per_sub + part * zrows, zrows)], z_v)
            pltpu.sync_copy(
                z_v, out_hbm.at[c, 1, pl.ds(s * per_sub + part * zrows, zrows)])

    return agg_kernel(hpa, hpb, row3, col3, w3)


def _tc_matmul(x, w):
    def body(x_ref, w_ref, o_ref):
        o_ref[...] = jnp.dot(x_ref[...], w_ref[...],
                             preferred_element_type=jnp.float32)

    return pl.pallas_call(
        body,
        out_shape=jax.ShapeDtypeStruct((x.shape[0], w.shape[1]), jnp.float32),
    )(x, w)


def _tc_scale(degp, h, n_nodes, n_pad):
    h2 = h.shape[1] // 2

    def body(degp_ref, h_ref, hpa_ref, hpb_ref, dis_ref):
        deg = (degp_ref[pl.ds(0, n_nodes)]
               + degp_ref[pl.ds(n_pad, n_nodes)] + 1.0)
        dis = lax.rsqrt(deg)
        dis_ref[...] = dis[:, None]
        hp = h_ref[...] * dis[:, None]
        hpa_ref[...] = hp[:, :h2]
        hpb_ref[...] = hp[:, h2:]

    return pl.pallas_call(
        body,
        out_shape=(
            jax.ShapeDtypeStruct((h.shape[0], h2), jnp.float32),
            jax.ShapeDtypeStruct((h.shape[0], h2), jnp.float32),
            jax.ShapeDtypeStruct((n_nodes, 1), jnp.float32),
        ),
    )(degp, h)


def _tc_layer_out(aggp, hpa, hpb, dis, b, n_nodes, w_next=None):
    h2 = hpa.shape[1]

    def make_out(aggp_ref, hpa_ref, hpb_ref, dis_ref, b_ref):
        agga = aggp_ref[0, 0, :n_nodes] + aggp_ref[1, 0, :n_nodes]
        aggb = aggp_ref[0, 1, :n_nodes] + aggp_ref[1, 1, :n_nodes]
        agg = jnp.concatenate([agga, aggb], axis=-1)
        hp = jnp.concatenate([hpa_ref[...], hpb_ref[...]], axis=-1)
        return jax.nn.relu(dis_ref[...] * (agg + hp) + b_ref[...])

    if w_next is None:
        def body(aggp_ref, hpa_ref, hpb_ref, dis_ref, b_ref, o_ref):
            o_ref[...] = make_out(aggp_ref, hpa_ref, hpb_ref, dis_ref, b_ref)

        return pl.pallas_call(
            body,
            out_shape=jax.ShapeDtypeStruct((n_nodes, 2 * h2), jnp.float32),
        )(aggp, hpa, hpb, dis, b[None, :])

    hn2 = w_next.shape[1] // 2

    def body2(aggp_ref, hpa_ref, hpb_ref, dis_ref, b_ref, w_ref,
              oa_ref, ob_ref):
        out = make_out(aggp_ref, hpa_ref, hpb_ref, dis_ref, b_ref)
        nxt = jnp.dot(out, w_ref[...],
                      preferred_element_type=jnp.float32) * dis_ref[...]
        oa_ref[...] = nxt[:, :hn2]
        ob_ref[...] = nxt[:, hn2:]

    return pl.pallas_call(
        body2,
        out_shape=(
            jax.ShapeDtypeStruct((n_nodes, hn2), jnp.float32),
            jax.ShapeDtypeStruct((n_nodes, hn2), jnp.float32),
        ),
    )(aggp, hpa, hpb, dis, b[None, :], w_next)


def kernel(x, edge_index, edge_weight, W0, b0, W1, b1):
    n_nodes = x.shape[0]
    n_edges = edge_weight.shape[0]
    n_chunks = -(-n_edges // (NW * K))
    n_chunks = -(-n_chunks // 4) * 4
    e_pad = NW * K * n_chunks

    row = edge_index[0].astype(jnp.int32)
    col = edge_index[1].astype(jnp.int32)
    w = edge_weight.astype(jnp.float32)
    pad = e_pad - n_edges
    if pad:
        row = jnp.concatenate([row, jnp.zeros((pad,), jnp.int32)])
        col = jnp.concatenate([col, jnp.zeros((pad,), jnp.int32)])
        w = jnp.concatenate([w, jnp.zeros((pad,), jnp.float32)])
    row3 = row.reshape(NW, n_chunks, K)
    col3 = col.reshape(NW, n_chunks, K)
    w3 = w.reshape(NW, n_chunks, K)

    degp = _sc_degree(row3, w3, n_nodes, n_chunks)
    h0 = _tc_matmul(x, W0)

    n_pad, _ = _pad_nodes(n_nodes)
    h0pa, h0pb, dis = _tc_scale(degp, h0, n_nodes, n_pad)
    agg0 = _sc_aggregate(h0pa, h0pb, row3, col3, w3, n_nodes, n_chunks,
                         W0.shape[1])
    h1pa, h1pb = _tc_layer_out(agg0, h0pa, h0pb, dis, b0, n_nodes, w_next=W1)
    agg1 = _sc_aggregate(h1pa, h1pb, row3, col3, w3, n_nodes, n_chunks,
                         W1.shape[1])
    return _tc_layer_out(agg1, h1pa, h1pb, dis, b1, n_nodes)
